# Initial kernel scaffold; baseline (speedup 1.0000x reference)
import jax
import jax.numpy as jnp
from jax.experimental import pallas as pl


def kernel(z, embeddings):
    raise NotImplementedError("write your pallas kernel here")


# speedup vs baseline: 1.4619x; 1.4619x over previous
import functools

import jax
import jax.numpy as jnp
from jax import lax
from jax.experimental import pallas as pl
from jax.experimental.pallas import tpu as pltpu
from jax.experimental.pallas import tpu_sc as plsc

KK = 8192
DD = 32
BETA = 0.25
ROWS = 256


def _argmin_body(sz_ref, se_ref, z_ref, et_ref, idx_ref, lsum_ref):
    m = lax.dot_general(
        z_ref[...], et_ref[...], (((1,), (0,)), ((), ())),
        preferred_element_type=jnp.float32)
    d = (sz_ref[...] + se_ref[...]) - 2.0 * m
    mind = jnp.min(d, axis=1, keepdims=True)
    kiota = lax.broadcasted_iota(jnp.int32, d.shape, 1)
    idx = jnp.min(jnp.where(d == mind, kiota, KK), axis=1)
    idx_ref[0, 0, :] = idx
    lsum_ref[0, 0, 0] = jnp.sum(mind)


def _tc_argmin(szf, se2, zf, et):
    bt = zf.shape[0]
    grid = bt // ROWS
    idx3, lsum = pl.pallas_call(
        _argmin_body,
        grid=(grid,),
        in_specs=[
            pl.BlockSpec((ROWS, 1), lambda g: (g, 0)),
            pl.BlockSpec((1, KK), lambda g: (0, 0)),
            pl.BlockSpec((ROWS, DD), lambda g: (g, 0)),
            pl.BlockSpec((DD, KK), lambda g: (0, 0)),
        ],
        out_specs=[
            pl.BlockSpec((1, 1, ROWS), lambda g: (g, 0, 0)),
            pl.BlockSpec((1, 1, 1), lambda g: (g, 0, 0), memory_space=pltpu.SMEM),
        ],
        out_shape=[
            jax.ShapeDtypeStruct((grid, 1, ROWS), jnp.int32),
            jax.ShapeDtypeStruct((grid, 1, 1), jnp.float32),
        ],
    )(szf, se2, zf, et)
    return idx3.reshape(bt), lsum


def _sc_gather(table, idx):
    bt = idx.shape[0]
    info = plsc.get_sparse_core_info()
    nw = info.num_cores * info.num_subcores
    b_per_w = bt // nw
    chunk = 128
    nchunk = b_per_w // chunk
    mesh = plsc.VectorSubcoreMesh(core_axis_name="c", subcore_axis_name="s")

    @functools.partial(
        pl.kernel,
        mesh=mesh,
        compiler_params=pltpu.CompilerParams(use_tc_tiling_on_sc=False),
        out_type=jax.ShapeDtypeStruct((bt, DD), jnp.float32),
        scratch_types=[
            pltpu.VMEM((b_per_w,), jnp.int32),
            pltpu.VMEM((b_per_w, DD), jnp.float32),
            pltpu.SemaphoreType.DMA,
        ],
    )
    def gather_kernel(table_hbm, idx_hbm, out_hbm, idx_v, rows_v, sem):
        wid = lax.axis_index("s") * info.num_cores + lax.axis_index("c")
        base = wid * b_per_w
        pltpu.sync_copy(idx_hbm.at[pl.ds(base, b_per_w)], idx_v)
        copies = []
        for j in range(nchunk):
            copies.append(pltpu.async_copy(
                table_hbm.at[idx_v.at[pl.ds(j * chunk, chunk)]],
                rows_v.at[pl.ds(j * chunk, chunk)], sem))
        for c in copies:
            c.wait()
        pltpu.sync_copy(rows_v, out_hbm.at[pl.ds(base, b_per_w)])

    return gather_kernel(table, idx)


def kernel(z, embeddings):
    b, t, d = z.shape
    bt = b * t
    s_z = (z ** 2).sum(axis=-1, keepdims=True)
    s_e = (embeddings ** 2).sum(axis=-1)
    zf = z.reshape(bt, d)
    szf = s_z.reshape(bt, 1)
    se2 = s_e.reshape(1, KK)
    et = embeddings.T

    closest_flat, lsum = _tc_argmin(szf, se2, zf, et)
    quantized_flat = _sc_gather(embeddings, closest_flat)

    loss = (1.0 + BETA) * (jnp.sum(lsum) / (bt * d))

    quantized_st = quantized_flat.reshape(b, t, d)
    closest = closest_flat.reshape(b, t)
    return quantized_st, loss, closest


# speedup vs baseline: 1.6696x; 1.1421x over previous
import functools

import jax
import jax.numpy as jnp
from jax import lax
from jax.experimental import pallas as pl
from jax.experimental.pallas import tpu as pltpu
from jax.experimental.pallas import tpu_sc as plsc

KK = 8192
DD = 32
BETA = 0.25
ROWS = 256


RSUB = 64
LCH = 128


def _argmin_body(sz_ref, se_ref, z_ref, et2_ref, idx_ref, lsum_ref):
    mm = lax.dot_general(
        z_ref[...], et2_ref[...], (((1,), (0,)), ((), ())),
        preferred_element_type=jnp.float32)
    sz = sz_ref[...]
    se = se_ref[...]
    nch = KK // LCH
    lane = lax.broadcasted_iota(jnp.int32, (RSUB, LCH), 1).astype(jnp.float32)
    psum = jnp.float32(0.0)
    idxs = []
    for r0 in range(0, ROWS, RSUB):
        szs = sz[r0:r0 + RSUB, :]
        runmin = (szs + se[:, 0:LCH]) + mm[r0:r0 + RSUB, 0:LCH]
        runcol = jnp.zeros((RSUB, LCH), jnp.int32)
        for j in range(1, nch):
            sl = slice(j * LCH, (j + 1) * LCH)
            dj = (szs + se[:, sl]) + mm[r0:r0 + RSUB, sl]
            mask = dj < runmin
            runmin = jnp.where(mask, dj, runmin)
            runcol = jnp.where(mask, j, runcol)
        rowmin = jnp.min(runmin, axis=1, keepdims=True)
        cand = runcol.astype(jnp.float32) * float(LCH) + lane
        kf = jnp.min(jnp.where(runmin == rowmin, cand, float(KK)), axis=1)
        idxs.append(kf.astype(jnp.int32))
        psum = psum + jnp.sum(rowmin)
    idx_ref[0, 0, :] = jnp.concatenate(idxs)
    lsum_ref[0, 0, 0] = psum


def _tc_argmin(sz2, se2, zf, et2):
    bt = zf.shape[0]
    grid = bt // ROWS
    idx3, lsum = pl.pallas_call(
        _argmin_body,
        grid=(grid,),
        in_specs=[
            pl.BlockSpec((ROWS, 1), lambda g: (g, 0)),
            pl.BlockSpec((1, KK), lambda g: (0, 0)),
            pl.BlockSpec((ROWS, DD), lambda g: (g, 0)),
            pl.BlockSpec((DD, KK), lambda g: (0, 0)),
        ],
        out_specs=[
            pl.BlockSpec((1, 1, ROWS), lambda g: (g, 0, 0)),
            pl.BlockSpec((1, 1, 1), lambda g: (g, 0, 0), memory_space=pltpu.SMEM),
        ],
        out_shape=[
            jax.ShapeDtypeStruct((grid, 1, ROWS), jnp.int32),
            jax.ShapeDtypeStruct((grid, 1, 1), jnp.float32),
        ],
    )(sz2, se2, zf, et2)
    return idx3.reshape(bt), lsum


def _sc_gather(table, idx):
    bt = idx.shape[0]
    info = plsc.get_sparse_core_info()
    nw = info.num_cores * info.num_subcores
    b_per_w = bt // nw
    chunk = 128
    nchunk = b_per_w // chunk
    mesh = plsc.VectorSubcoreMesh(core_axis_name="c", subcore_axis_name="s")

    @functools.partial(
        pl.kernel,
        mesh=mesh,
        compiler_params=pltpu.CompilerParams(use_tc_tiling_on_sc=False),
        out_type=jax.ShapeDtypeStruct((bt, DD), jnp.float32),
        scratch_types=[
            pltpu.VMEM((b_per_w,), jnp.int32),
            pltpu.VMEM((b_per_w, DD), jnp.float32),
            pltpu.SemaphoreType.DMA,
        ],
    )
    def gather_kernel(table_hbm, idx_hbm, out_hbm, idx_v, rows_v, sem):
        wid = lax.axis_index("s") * info.num_cores + lax.axis_index("c")
        base = wid * b_per_w
        pltpu.sync_copy(idx_hbm.at[pl.ds(base, b_per_w)], idx_v)
        copies = []
        for j in range(nchunk):
            copies.append(pltpu.async_copy(
                table_hbm.at[idx_v.at[pl.ds(j * chunk, chunk)]],
                rows_v.at[pl.ds(j * chunk, chunk)], sem))
        for c in copies:
            c.wait()
        pltpu.sync_copy(rows_v, out_hbm.at[pl.ds(base, b_per_w)])

    return gather_kernel(table, idx)


def kernel(z, embeddings):
    b, t, d = z.shape
    bt = b * t
    s_z = (z ** 2).sum(axis=-1, keepdims=True)
    s_e = (embeddings ** 2).sum(axis=-1)
    zf = z.reshape(bt, d)
    zb = zf.astype(jnp.bfloat16)
    et2 = (-2.0 * embeddings.T).astype(jnp.bfloat16)

    closest_flat, lsum = _tc_argmin(
        s_z.reshape(bt, 1), s_e.reshape(1, KK), zb, et2)
    tbl = embeddings.astype(jnp.bfloat16).astype(jnp.float32)
    quantized_flat = _sc_gather(tbl, closest_flat)

    loss = (1.0 + BETA) * (jnp.sum(lsum) / (bt * d))

    quantized_st = (zf + (quantized_flat - zf)).reshape(b, t, d)
    closest = closest_flat.reshape(b, t)
    return quantized_st, loss, closest
